# baseline (speedup 1.0000x reference)
import functools

import jax
import jax.numpy as jnp
import numpy as np
from jax import lax
from jax.experimental import pallas as pl
from jax.experimental.pallas import tpu as pltpu
from jax.experimental.pallas import tpu_sc as plsc

N_NODES_C = 30000
N_EDGES_C = 60000
N_MOL_C = 1200

_NC, _NS, _NW = 2, 16, 32
_CH = 128

_E_PAD = 61440
_N_PAD = 32768
_NSEG_NODE = 30720
_NSEG_MOL = 1280

_R_NP = np.kron(np.eye(32, dtype=np.float32), np.ones((1, 32), dtype=np.float32))
_T_NP = np.kron(np.ones((1, 32), dtype=np.float32), np.eye(32, dtype=np.float32))


def _sigmoid(v):
    return 1.0 / (1.0 + jnp.exp(-v))


def _silu(v):
    return v * _sigmoid(v)


def _mesh():
    return plsc.VectorSubcoreMesh(core_axis_name="c", subcore_axis_name="s",
                                  num_cores=_NC, num_subcores=_NS)


def _sc_gather(table, idx):
    e_pad = idx.shape[0]
    epw = e_pad // _NW
    nchunk = epw // _CH
    nbuf = 4

    @functools.partial(
        pl.kernel,
        out_type=jax.ShapeDtypeStruct((e_pad, 128), jnp.float32),
        mesh=_mesh(),
        scratch_types=[
            pltpu.VMEM((epw,), jnp.int32),
            pltpu.VMEM((nbuf, _CH, 128), jnp.float32),
            pltpu.SemaphoreType.DMA,
        ],
        compiler_params=pltpu.CompilerParams(use_tc_tiling_on_sc=False),
    )
    def k(table_hbm, idx_hbm, out_hbm, idx_v, rows_v, sem):
        c = lax.axis_index("c")
        s = lax.axis_index("s")
        base = (c * _NS + s) * epw
        pltpu.sync_copy(idx_hbm.at[pl.ds(base, epw)], idx_v)
        cps = {}
        for j in range(min(nbuf, nchunk)):
            cps[j] = pltpu.async_copy(
                table_hbm.at[idx_v.at[pl.ds(j * _CH, _CH)]],
                rows_v.at[j % nbuf], sem)
        for j in range(nchunk):
            cps[j].wait()
            pltpu.sync_copy(rows_v.at[j % nbuf],
                            out_hbm.at[pl.ds(base + j * _CH, _CH)])
            nxt = j + nbuf
            if nxt < nchunk:
                cps[nxt] = pltpu.async_copy(
                    table_hbm.at[idx_v.at[pl.ds(nxt * _CH, _CH)]],
                    rows_v.at[nxt % nbuf], sem)

    return k(table, idx)


def _sc_scatter_add(data, idx3, zeros, nseg_pad):
    nchunk = idx3.shape[1]
    epw = nchunk * _CH
    stripe = nseg_pad // _NS

    @functools.partial(
        pl.kernel,
        out_type=jax.ShapeDtypeStruct((_NC, nseg_pad, 32), jnp.float32),
        mesh=_mesh(),
        scratch_types=[
            pltpu.VMEM((nchunk, _CH), jnp.int32),
            pltpu.VMEM((epw, 32), jnp.float32),
            pltpu.VMEM_SHARED((nseg_pad, 32), jnp.float32),
            pltpu.SemaphoreType.DMA,
        ],
        compiler_params=pltpu.CompilerParams(use_tc_tiling_on_sc=False),
    )
    def k(data_hbm, idx_hbm, zeros_hbm, out_hbm, idx_v, data_v, acc_sh, sem):
        c = lax.axis_index("c")
        s = lax.axis_index("s")
        wid = c * _NS + s
        pltpu.sync_copy(zeros_hbm.at[pl.ds(s * stripe, stripe)],
                        acc_sh.at[pl.ds(s * stripe, stripe)])
        pltpu.sync_copy(idx_hbm.at[wid], idx_v)
        pltpu.sync_copy(data_hbm.at[pl.ds(wid * epw, epw)], data_v)
        plsc.subcore_barrier()
        cps = []
        for j in range(nchunk):
            cps.append(pltpu.async_copy(
                data_v.at[pl.ds(j * _CH, _CH)], acc_sh.at[idx_v.at[j]],
                sem, add=True))
        for cp in cps:
            cp.wait()
        plsc.subcore_barrier()
        pltpu.sync_copy(acc_sh.at[pl.ds(s * stripe, stripe)],
                        out_hbm.at[c, pl.ds(s * stripe, stripe)])

    return k(data, idx3, zeros)


def _msg_body(ea_ref, xs_ref, wa_ref, ba_ref, wb2_ref, bb_ref, r_ref, t_ref,
              out_ref):
    h = jnp.maximum(ea_ref[...] @ wa_ref[...] + ba_ref[...], 0.0)
    xs = xs_ref[:, 0:32]
    a = h @ r_ref[...]
    xs4 = jnp.concatenate([xs, xs, xs, xs], axis=1)
    b = jnp.concatenate([xs4] * 8, axis=1)
    z = a * b
    out_ref[...] = z @ wb2_ref[...] + xs @ bb_ref[...]


def _msg(edge_attr, x_src, wa, ba, wb2, bb, block_e=2000):
    e = edge_attr.shape[0]
    e_out = x_src.shape[0]
    grid = e // block_e
    full = lambda shp: pl.BlockSpec(shp, lambda i: (0, 0))
    return pl.pallas_call(
        _msg_body,
        grid=(grid,),
        in_specs=[
            pl.BlockSpec((block_e, 16), lambda i: (i, 0)),
            pl.BlockSpec((block_e, 128), lambda i: (i, 0)),
            full((16, 32)), full((1, 32)), full((1024, 32)), full((32, 32)),
            full((32, 1024)), full((32, 1024)),
        ],
        out_specs=pl.BlockSpec((block_e, 32), lambda i: (i, 0)),
        out_shape=jax.ShapeDtypeStruct((e_out, 32), jnp.float32),
    )(edge_attr, x_src, wa, ba.reshape(1, 32), wb2, bb,
      jnp.asarray(_R_NP), jnp.asarray(_T_NP))


def _upd_body(a0_ref, a1_ref, x_ref, root_ref, bias_ref, out_ref, *, wide):
    y = a0_ref[0] + a1_ref[0] + x_ref[:, 0:32] @ root_ref[...] + bias_ref[...]
    v = _silu(y)
    if wide:
        v = jnp.concatenate([v, jnp.zeros((v.shape[0], 96), v.dtype)], axis=1)
    out_ref[...] = v


def _update(aggp, x, root, bias, n_out, wide, block_n=3000):
    n = min(x.shape[0], N_NODES_C)
    grid = n // block_n
    w_out = 128 if wide else 32
    full = lambda shp: pl.BlockSpec(shp, lambda i: (0,) * len(shp))
    return pl.pallas_call(
        functools.partial(_upd_body, wide=wide),
        grid=(grid,),
        in_specs=[
            pl.BlockSpec((1, block_n, 32), lambda i: (0, i, 0)),
            pl.BlockSpec((1, block_n, 32), lambda i: (1, i, 0)),
            pl.BlockSpec((block_n, 128), lambda i: (i, 0)),
            full((32, 32)), full((1, 32)),
        ],
        out_specs=pl.BlockSpec((block_n, w_out), lambda i: (i, 0)),
        out_shape=jax.ShapeDtypeStruct((n_out, w_out), jnp.float32),
    )(aggp, aggp, x, root, bias.reshape(1, 32))


def _y_stats_body(a0_ref, a1_ref, x_ref, root_ref, bias_ref, y_ref, stats_ref):
    y = a0_ref[0] + a1_ref[0] + x_ref[:, 0:32] @ root_ref[...] + bias_ref[...]
    y_ref[...] = y

    @pl.when(pl.program_id(0) == 0)
    def _():
        stats_ref[...] = jnp.zeros_like(stats_ref)

    stats_ref[0:1, :] += jnp.sum(y, axis=0, keepdims=True)
    stats_ref[1:2, :] += jnp.sum(y * y, axis=0, keepdims=True)


def _bn_apply_body(y_ref, stats_ref, gamma_ref, beta_ref, out_ref, *, n):
    mean = stats_ref[0:1, :] / n
    var = stats_ref[1:2, :] / n - mean * mean
    yn = (y_ref[...] - mean) * jax.lax.rsqrt(var + 1e-5) * gamma_ref[...] \
        + beta_ref[...]
    v = _silu(yn)
    out_ref[...] = jnp.concatenate(
        [v, jnp.zeros((v.shape[0], 96), v.dtype)], axis=1)


def _update_bn(aggp, x, root, bias, gamma, beta, block_n=3000):
    n = x.shape[0]
    grid = n // block_n
    full = lambda shp: pl.BlockSpec(shp, lambda i: (0,) * len(shp))
    y, stats = pl.pallas_call(
        _y_stats_body,
        grid=(grid,),
        in_specs=[
            pl.BlockSpec((1, block_n, 32), lambda i: (0, i, 0)),
            pl.BlockSpec((1, block_n, 32), lambda i: (1, i, 0)),
            pl.BlockSpec((block_n, 128), lambda i: (i, 0)),
            full((32, 32)), full((1, 32)),
        ],
        out_specs=[
            pl.BlockSpec((block_n, 32), lambda i: (i, 0)),
            pl.BlockSpec((2, 32), lambda i: (0, 0)),
        ],
        out_shape=[
            jax.ShapeDtypeStruct((n, 32), jnp.float32),
            jax.ShapeDtypeStruct((2, 32), jnp.float32),
        ],
    )(aggp, aggp, x, root, bias.reshape(1, 32))
    return pl.pallas_call(
        functools.partial(_bn_apply_body, n=float(n)),
        grid=(grid,),
        in_specs=[
            pl.BlockSpec((block_n, 32), lambda i: (i, 0)),
            full((2, 32)), full((1, 32)), full((1, 32)),
        ],
        out_specs=pl.BlockSpec((block_n, 128), lambda i: (i, 0)),
        out_shape=jax.ShapeDtypeStruct((n, 128), jnp.float32),
    )(y, stats, gamma.reshape(1, 32), beta.reshape(1, 32))


def _head_body(pp_ref, molf_ref, wm0_ref, bm0_ref, gm_ref, btm_ref,
               wm1_ref, bm1_ref, wf0_ref, bf0_ref, wf1_ref, bf1_ref,
               wf2_ref, bf2_ref, wf3_ref, bf3_ref, out_ref):
    pool = pp_ref[0, 0:N_MOL_C, :] + pp_ref[1, 0:N_MOL_C, :]
    hm = molf_ref[...] @ wm0_ref[...] + bm0_ref[...]
    m = jnp.mean(hm, axis=0, keepdims=True)
    v = jnp.mean((hm - m) * (hm - m), axis=0, keepdims=True)
    hm = _silu((hm - m) * jax.lax.rsqrt(v + 1e-5) * gm_ref[...] + btm_ref[...])
    hm = _silu(hm @ wm1_ref[...] + bm1_ref[...])
    hg = jnp.concatenate([pool, hm], axis=1)
    hg = _silu(hg @ wf0_ref[...] + bf0_ref[...])
    hg = _silu(hg @ wf1_ref[...] + bf1_ref[...])
    hg = _silu(hg @ wf2_ref[...] + bf2_ref[...])
    out_ref[...] = hg @ wf3_ref[...] + bf3_ref[...]


def _head(poolp, mol_feats, p):
    wf3p = jnp.pad(p['Wf3'], ((0, 0), (0, 127)))
    bf3p = jnp.pad(p['bf3'].reshape(1, 1), ((0, 0), (0, 127)))
    args = (poolp, mol_feats,
            p['Wm0'], p['bm0'].reshape(1, 32),
            p['gamma_m'].reshape(1, 32), p['beta_m'].reshape(1, 32),
            p['Wm1'], p['bm1'].reshape(1, 32),
            p['Wf0'], p['bf0'].reshape(1, 128),
            p['Wf1'], p['bf1'].reshape(1, 128),
            p['Wf2'], p['bf2'].reshape(1, 64),
            wf3p, bf3p)
    in_specs = [pl.BlockSpec(a.shape, functools.partial(lambda nd: (0,) * nd,
                                                        a.ndim))
                for a in args]
    out = pl.pallas_call(
        _head_body,
        in_specs=in_specs,
        out_specs=pl.BlockSpec((N_MOL_C, 128), lambda: (0, 0)),
        out_shape=jax.ShapeDtypeStruct((N_MOL_C, 128), jnp.float32),
    )(*args)
    return out[:, :1]


def kernel(x, edge_attr, mol_feats, params, edge_index, batch):
    p = params
    n = x.shape[0]
    e = edge_attr.shape[0]

    wb2_1 = p['W1b'].reshape(1024, 32)
    bb_1 = p['b1b'].reshape(32, 32)
    wb2_2 = p['W2b'].reshape(1024, 32)
    bb_2 = p['b2b'].reshape(32, 32)

    src_pad = jnp.pad(edge_index[0], (0, _E_PAD - e))
    dst3 = jnp.pad(edge_index[1], (0, _E_PAD - e), constant_values=n) \
        .reshape(_NW, -1, _CH)
    batch3 = jnp.pad(batch, (0, _N_PAD - n), constant_values=N_MOL_C) \
        .reshape(_NW, -1, _CH)
    zeros_node = jnp.zeros((_NSEG_NODE, 32), jnp.float32)
    zeros_mol = jnp.zeros((_NSEG_MOL, 32), jnp.float32)

    def conv(h_nodes, wa, ba, wb2, bb):
        hs = _sc_gather(h_nodes, src_pad)
        msg = _msg(edge_attr, hs, wa, ba, wb2, bb)
        return _sc_scatter_add(msg, dst3, zeros_node, _NSEG_NODE)

    xp = jnp.pad(x, ((0, 0), (0, 96)))

    aggp = conv(xp, p['W1a'], p['b1a'], wb2_1, bb_1)
    h = _update_bn(aggp, xp, p['root0'], p['bias0'], p['gamma_gc'],
                   p['beta_gc'])

    aggp = conv(h, p['W2a'], p['b2a'], wb2_2, bb_2)
    h2 = _update(aggp, h, p['root1'], p['bias1'], n, wide=True)

    aggp = conv(h2, p['W2a'], p['b2a'], wb2_2, bb_2)
    h3p = _update(aggp, h2, p['root2'], p['bias2'], _N_PAD, wide=False)

    poolp = _sc_scatter_add(h3p, batch3, zeros_mol, _NSEG_MOL)
    return _head(poolp, mol_feats, p)


# speedup vs baseline: 1.1492x; 1.1492x over previous
import functools

import jax
import jax.numpy as jnp
import numpy as np
from jax import lax
from jax.experimental import pallas as pl
from jax.experimental.pallas import tpu as pltpu
from jax.experimental.pallas import tpu_sc as plsc

N_NODES_C = 30000
N_EDGES_C = 60000
N_MOL_C = 1200

_NC, _NS, _NW = 2, 16, 32
_CH = 128

_E_PAD = 61440
_N_PAD = 32768
_NSEG_NODE = 30720
_NSEG_MOL = 1280

_R_NP = np.kron(np.eye(32, dtype=np.float32), np.ones((1, 32), dtype=np.float32))
_T_NP = np.kron(np.ones((1, 32), dtype=np.float32), np.eye(32, dtype=np.float32))


def _sigmoid(v):
    return 1.0 / (1.0 + jnp.exp(-v))


def _silu(v):
    return v * _sigmoid(v)


def _mesh():
    return plsc.VectorSubcoreMesh(core_axis_name="c", subcore_axis_name="s",
                                  num_cores=_NC, num_subcores=_NS)


def _sc_gather(table, idx):
    e_pad = idx.shape[0]
    epw = e_pad // _NW
    nchunk = epw // _CH

    @functools.partial(
        pl.kernel,
        out_type=jax.ShapeDtypeStruct((e_pad, 32), jnp.float32),
        mesh=_mesh(),
        scratch_types=[
            pltpu.VMEM((epw,), jnp.int32),
            pltpu.VMEM((epw, 32), jnp.float32),
            pltpu.SemaphoreType.DMA,
        ],
        compiler_params=pltpu.CompilerParams(use_tc_tiling_on_sc=False),
    )
    def k(table_hbm, idx_hbm, out_hbm, idx_v, rows_v, sem):
        c = lax.axis_index("c")
        s = lax.axis_index("s")
        base = (c * _NS + s) * epw
        pltpu.sync_copy(idx_hbm.at[pl.ds(base, epw)], idx_v)
        cps = []
        for j in range(nchunk):
            cps.append(pltpu.async_copy(
                table_hbm.at[idx_v.at[pl.ds(j * _CH, _CH)]],
                rows_v.at[pl.ds(j * _CH, _CH)], sem))
        for cp in cps:
            cp.wait()
        pltpu.sync_copy(rows_v, out_hbm.at[pl.ds(base, epw)])

    return k(table, idx)


def _sc_scatter_add(data, idx3, zeros, nseg_pad):
    nchunk = idx3.shape[1]
    epw = nchunk * _CH
    stripe = nseg_pad // _NS

    @functools.partial(
        pl.kernel,
        out_type=jax.ShapeDtypeStruct((_NC, nseg_pad, 32), jnp.float32),
        mesh=_mesh(),
        scratch_types=[
            pltpu.VMEM((nchunk, _CH), jnp.int32),
            pltpu.VMEM((epw, 32), jnp.float32),
            pltpu.VMEM_SHARED((nseg_pad, 32), jnp.float32),
            pltpu.SemaphoreType.DMA,
        ],
        compiler_params=pltpu.CompilerParams(use_tc_tiling_on_sc=False),
    )
    def k(data_hbm, idx_hbm, zeros_hbm, out_hbm, idx_v, data_v, acc_sh, sem):
        c = lax.axis_index("c")
        s = lax.axis_index("s")
        wid = c * _NS + s
        pltpu.sync_copy(zeros_hbm.at[pl.ds(s * stripe, stripe)],
                        acc_sh.at[pl.ds(s * stripe, stripe)])
        pltpu.sync_copy(idx_hbm.at[wid], idx_v)
        pltpu.sync_copy(data_hbm.at[pl.ds(wid * epw, epw)], data_v)
        plsc.subcore_barrier()
        cps = []
        for j in range(nchunk):
            cps.append(pltpu.async_copy(
                data_v.at[pl.ds(j * _CH, _CH)], acc_sh.at[idx_v.at[j]],
                sem, add=True))
        for cp in cps:
            cp.wait()
        plsc.subcore_barrier()
        pltpu.sync_copy(acc_sh.at[pl.ds(s * stripe, stripe)],
                        out_hbm.at[c, pl.ds(s * stripe, stripe)])

    return k(data, idx3, zeros)


def _msg_body(ea_ref, xs_ref, wa_ref, ba_ref, wb2_ref, bb_ref, r_ref, t_ref,
              out_ref):
    h = jnp.maximum(ea_ref[...] @ wa_ref[...] + ba_ref[...], 0.0)
    xs = xs_ref[...]
    a = h @ r_ref[...]
    xs4 = jnp.concatenate([xs, xs, xs, xs], axis=1)
    b = jnp.concatenate([xs4] * 8, axis=1)
    z = a * b
    out_ref[...] = z @ wb2_ref[...] + xs @ bb_ref[...]


def _msg(edge_attr, x_src, wa, ba, wb2, bb, block_e=2000):
    e = edge_attr.shape[0]
    e_out = x_src.shape[0]
    grid = e // block_e
    full = lambda shp: pl.BlockSpec(shp, lambda i: (0, 0))
    return pl.pallas_call(
        _msg_body,
        grid=(grid,),
        in_specs=[
            pl.BlockSpec((block_e, 16), lambda i: (i, 0)),
            pl.BlockSpec((block_e, 32), lambda i: (i, 0)),
            full((16, 32)), full((1, 32)), full((1024, 32)), full((32, 32)),
            full((32, 1024)), full((32, 1024)),
        ],
        out_specs=pl.BlockSpec((block_e, 32), lambda i: (i, 0)),
        out_shape=jax.ShapeDtypeStruct((e_out, 32), jnp.float32),
    )(edge_attr, x_src, wa, ba.reshape(1, 32), wb2, bb,
      jnp.asarray(_R_NP), jnp.asarray(_T_NP))


def _upd_body(a0_ref, a1_ref, x_ref, root_ref, bias_ref, out_ref):
    y = a0_ref[0] + a1_ref[0] + x_ref[...] @ root_ref[...] + bias_ref[...]
    out_ref[...] = _silu(y)


def _update(aggp, x, root, bias, n_out, block_n=3000):
    n = x.shape[0]
    grid = n // block_n
    full = lambda shp: pl.BlockSpec(shp, lambda i: (0,) * len(shp))
    return pl.pallas_call(
        _upd_body,
        grid=(grid,),
        in_specs=[
            pl.BlockSpec((1, block_n, 32), lambda i: (0, i, 0)),
            pl.BlockSpec((1, block_n, 32), lambda i: (1, i, 0)),
            pl.BlockSpec((block_n, 32), lambda i: (i, 0)),
            full((32, 32)), full((1, 32)),
        ],
        out_specs=pl.BlockSpec((block_n, 32), lambda i: (i, 0)),
        out_shape=jax.ShapeDtypeStruct((n_out, 32), jnp.float32),
    )(aggp, aggp, x, root, bias.reshape(1, 32))


def _y_stats_body(a0_ref, a1_ref, x_ref, root_ref, bias_ref, y_ref, stats_ref):
    y = a0_ref[0] + a1_ref[0] + x_ref[...] @ root_ref[...] + bias_ref[...]
    y_ref[...] = y

    @pl.when(pl.program_id(0) == 0)
    def _():
        stats_ref[...] = jnp.zeros_like(stats_ref)

    stats_ref[0:1, :] += jnp.sum(y, axis=0, keepdims=True)
    stats_ref[1:2, :] += jnp.sum(y * y, axis=0, keepdims=True)


def _bn_apply_body(y_ref, stats_ref, gamma_ref, beta_ref, out_ref, *, n):
    mean = stats_ref[0:1, :] / n
    var = stats_ref[1:2, :] / n - mean * mean
    yn = (y_ref[...] - mean) * jax.lax.rsqrt(var + 1e-5) * gamma_ref[...] \
        + beta_ref[...]
    out_ref[...] = _silu(yn)


def _update_bn(aggp, x, root, bias, gamma, beta, block_n=3000):
    n = x.shape[0]
    grid = n // block_n
    full = lambda shp: pl.BlockSpec(shp, lambda i: (0,) * len(shp))
    y, stats = pl.pallas_call(
        _y_stats_body,
        grid=(grid,),
        in_specs=[
            pl.BlockSpec((1, block_n, 32), lambda i: (0, i, 0)),
            pl.BlockSpec((1, block_n, 32), lambda i: (1, i, 0)),
            pl.BlockSpec((block_n, 32), lambda i: (i, 0)),
            full((32, 32)), full((1, 32)),
        ],
        out_specs=[
            pl.BlockSpec((block_n, 32), lambda i: (i, 0)),
            pl.BlockSpec((2, 32), lambda i: (0, 0)),
        ],
        out_shape=[
            jax.ShapeDtypeStruct((n, 32), jnp.float32),
            jax.ShapeDtypeStruct((2, 32), jnp.float32),
        ],
    )(aggp, aggp, x, root, bias.reshape(1, 32))
    return pl.pallas_call(
        functools.partial(_bn_apply_body, n=float(n)),
        grid=(grid,),
        in_specs=[
            pl.BlockSpec((block_n, 32), lambda i: (i, 0)),
            full((2, 32)), full((1, 32)), full((1, 32)),
        ],
        out_specs=pl.BlockSpec((block_n, 32), lambda i: (i, 0)),
        out_shape=jax.ShapeDtypeStruct((n, 32), jnp.float32),
    )(y, stats, gamma.reshape(1, 32), beta.reshape(1, 32))


def _head_body(pp_ref, molf_ref, wm0_ref, bm0_ref, gm_ref, btm_ref,
               wm1_ref, bm1_ref, wf0_ref, bf0_ref, wf1_ref, bf1_ref,
               wf2_ref, bf2_ref, wf3_ref, bf3_ref, out_ref):
    pool = pp_ref[0, 0:N_MOL_C, :] + pp_ref[1, 0:N_MOL_C, :]
    hm = molf_ref[...] @ wm0_ref[...] + bm0_ref[...]
    m = jnp.mean(hm, axis=0, keepdims=True)
    v = jnp.mean((hm - m) * (hm - m), axis=0, keepdims=True)
    hm = _silu((hm - m) * jax.lax.rsqrt(v + 1e-5) * gm_ref[...] + btm_ref[...])
    hm = _silu(hm @ wm1_ref[...] + bm1_ref[...])
    hg = jnp.concatenate([pool, hm], axis=1)
    hg = _silu(hg @ wf0_ref[...] + bf0_ref[...])
    hg = _silu(hg @ wf1_ref[...] + bf1_ref[...])
    hg = _silu(hg @ wf2_ref[...] + bf2_ref[...])
    out_ref[...] = hg @ wf3_ref[...] + bf3_ref[...]


def _head(poolp, mol_feats, p):
    wf3p = jnp.pad(p['Wf3'], ((0, 0), (0, 127)))
    bf3p = jnp.pad(p['bf3'].reshape(1, 1), ((0, 0), (0, 127)))
    args = (poolp, mol_feats,
            p['Wm0'], p['bm0'].reshape(1, 32),
            p['gamma_m'].reshape(1, 32), p['beta_m'].reshape(1, 32),
            p['Wm1'], p['bm1'].reshape(1, 32),
            p['Wf0'], p['bf0'].reshape(1, 128),
            p['Wf1'], p['bf1'].reshape(1, 128),
            p['Wf2'], p['bf2'].reshape(1, 64),
            wf3p, bf3p)
    in_specs = [pl.BlockSpec(a.shape, functools.partial(lambda nd: (0,) * nd,
                                                        a.ndim))
                for a in args]
    out = pl.pallas_call(
        _head_body,
        in_specs=in_specs,
        out_specs=pl.BlockSpec((N_MOL_C, 128), lambda: (0, 0)),
        out_shape=jax.ShapeDtypeStruct((N_MOL_C, 128), jnp.float32),
    )(*args)
    return out[:, :1]


def kernel(x, edge_attr, mol_feats, params, edge_index, batch):
    p = params
    n = x.shape[0]
    e = edge_attr.shape[0]

    wb2_1 = p['W1b'].reshape(1024, 32)
    bb_1 = p['b1b'].reshape(32, 32)
    wb2_2 = p['W2b'].reshape(1024, 32)
    bb_2 = p['b2b'].reshape(32, 32)

    src_pad = jnp.pad(edge_index[0], (0, _E_PAD - e))
    dst3 = jnp.pad(edge_index[1], (0, _E_PAD - e), constant_values=n) \
        .reshape(_NW, -1, _CH)
    batch3 = jnp.pad(batch, (0, _N_PAD - n), constant_values=N_MOL_C) \
        .reshape(_NW, -1, _CH)
    zeros_node = jnp.zeros((_NSEG_NODE, 32), jnp.float32)
    zeros_mol = jnp.zeros((_NSEG_MOL, 32), jnp.float32)

    def conv(h_nodes, wa, ba, wb2, bb):
        hs = _sc_gather(h_nodes, src_pad)
        msg = _msg(edge_attr, hs, wa, ba, wb2, bb)
        return _sc_scatter_add(msg, dst3, zeros_node, _NSEG_NODE)

    aggp = conv(x, p['W1a'], p['b1a'], wb2_1, bb_1)
    h = _update_bn(aggp, x, p['root0'], p['bias0'], p['gamma_gc'], p['beta_gc'])

    aggp = conv(h, p['W2a'], p['b2a'], wb2_2, bb_2)
    h2 = _update(aggp, h, p['root1'], p['bias1'], n)

    aggp = conv(h2, p['W2a'], p['b2a'], wb2_2, bb_2)
    h3p = _update(aggp, h2, p['root2'], p['bias2'], _N_PAD)

    poolp = _sc_scatter_add(h3p, batch3, zeros_mol, _NSEG_MOL)
    return _head(poolp, mol_feats, p)


# speedup vs baseline: 1.1508x; 1.0014x over previous
import functools

import jax
import jax.numpy as jnp
import numpy as np
from jax import lax
from jax.experimental import pallas as pl
from jax.experimental.pallas import tpu as pltpu
from jax.experimental.pallas import tpu_sc as plsc

N_NODES_C = 30000
N_EDGES_C = 60000
N_MOL_C = 1200

_NC, _NS, _NW = 2, 16, 32
_CH = 128

_E_PAD = 61440
_N_PAD = 32768
_NSEG_NODE = 30720
_NSEG_MOL = 1280

_R_NP = np.kron(np.eye(32, dtype=np.float32), np.ones((1, 32), dtype=np.float32))


def _sigmoid(v):
    return 1.0 / (1.0 + jnp.exp(-v))


def _silu(v):
    return v * _sigmoid(v)


def _mesh():
    return plsc.VectorSubcoreMesh(core_axis_name="c", subcore_axis_name="s",
                                  num_cores=_NC, num_subcores=_NS)


def _sc_gather(table, idx):
    e_pad = idx.shape[0]
    epw = e_pad // _NW
    nchunk = epw // _CH

    @functools.partial(
        pl.kernel,
        out_type=jax.ShapeDtypeStruct((e_pad, 32), jnp.float32),
        mesh=_mesh(),
        scratch_types=[
            pltpu.VMEM((epw,), jnp.int32),
            pltpu.VMEM((epw, 32), jnp.float32),
            pltpu.SemaphoreType.DMA,
        ],
        compiler_params=pltpu.CompilerParams(use_tc_tiling_on_sc=False),
    )
    def k(table_hbm, idx_hbm, out_hbm, idx_v, rows_v, sem):
        c = lax.axis_index("c")
        s = lax.axis_index("s")
        base = (c * _NS + s) * epw
        pltpu.sync_copy(idx_hbm.at[pl.ds(base, epw)], idx_v)
        cps = []
        for j in range(nchunk):
            cps.append(pltpu.async_copy(
                table_hbm.at[idx_v.at[pl.ds(j * _CH, _CH)]],
                rows_v.at[pl.ds(j * _CH, _CH)], sem))
        for cp in cps:
            cp.wait()
        pltpu.sync_copy(rows_v, out_hbm.at[pl.ds(base, epw)])

    return k(table, idx)


def _sc_scatter_add(data, idx3, zeros, nseg_pad):
    nchunk = idx3.shape[1]
    epw = nchunk * _CH
    stripe = nseg_pad // _NS

    @functools.partial(
        pl.kernel,
        out_type=jax.ShapeDtypeStruct((_NC, nseg_pad, 32), jnp.float32),
        mesh=_mesh(),
        scratch_types=[
            pltpu.VMEM((nchunk, _CH), jnp.int32),
            pltpu.VMEM((epw, 32), jnp.float32),
            pltpu.VMEM_SHARED((nseg_pad, 32), jnp.float32),
            pltpu.SemaphoreType.DMA,
        ],
        compiler_params=pltpu.CompilerParams(use_tc_tiling_on_sc=False),
    )
    def k(data_hbm, idx_hbm, zeros_hbm, out_hbm, idx_v, data_v, acc_sh, sem):
        c = lax.axis_index("c")
        s = lax.axis_index("s")
        wid = c * _NS + s
        pltpu.sync_copy(zeros_hbm.at[pl.ds(s * stripe, stripe)],
                        acc_sh.at[pl.ds(s * stripe, stripe)])
        pltpu.sync_copy(idx_hbm.at[wid], idx_v)
        pltpu.sync_copy(data_hbm.at[pl.ds(wid * epw, epw)], data_v)
        plsc.subcore_barrier()
        cps = []
        for j in range(nchunk):
            cps.append(pltpu.async_copy(
                data_v.at[pl.ds(j * _CH, _CH)], acc_sh.at[idx_v.at[j]],
                sem, add=True))
        for cp in cps:
            cp.wait()
        plsc.subcore_barrier()
        pltpu.sync_copy(acc_sh.at[pl.ds(s * stripe, stripe)],
                        out_hbm.at[c, pl.ds(s * stripe, stripe)])

    return k(data, idx3, zeros)


def _msg_body(ea_ref, xs_ref, wa_ref, ba_ref, wb2_ref, bb_ref, r_ref,
              out_ref):
    h = jnp.maximum(ea_ref[...] @ wa_ref[...] + ba_ref[...], 0.0)
    xs = xs_ref[...]
    a = h @ r_ref[...]
    xs4 = jnp.concatenate([xs, xs, xs, xs], axis=1)
    b = jnp.concatenate([xs4] * 8, axis=1)
    z = a * b
    out_ref[...] = z @ wb2_ref[...] + xs @ bb_ref[...]


def _msg(edge_attr, x_src, wa, ba, wb2, bb, block_e=2000):
    e = edge_attr.shape[0]
    e_out = x_src.shape[0]
    grid = e // block_e
    full = lambda shp: pl.BlockSpec(shp, lambda i: (0, 0))
    return pl.pallas_call(
        _msg_body,
        grid=(grid,),
        in_specs=[
            pl.BlockSpec((block_e, 16), lambda i: (i, 0)),
            pl.BlockSpec((block_e, 32), lambda i: (i, 0)),
            full((16, 32)), full((1, 32)), full((1024, 32)), full((32, 32)),
            full((32, 1024)),
        ],
        out_specs=pl.BlockSpec((block_e, 32), lambda i: (i, 0)),
        out_shape=jax.ShapeDtypeStruct((e_out, 32), jnp.float32),
    )(edge_attr, x_src, wa, ba.reshape(1, 32), wb2, bb,
      jnp.asarray(_R_NP))


def _upd_body(a0_ref, a1_ref, x_ref, root_ref, bias_ref, out_ref):
    y = a0_ref[0] + a1_ref[0] + x_ref[...] @ root_ref[...] + bias_ref[...]
    out_ref[...] = _silu(y)


def _update(aggp, x, root, bias, n_out, block_n=3000):
    n = x.shape[0]
    grid = n // block_n
    full = lambda shp: pl.BlockSpec(shp, lambda i: (0,) * len(shp))
    return pl.pallas_call(
        _upd_body,
        grid=(grid,),
        in_specs=[
            pl.BlockSpec((1, block_n, 32), lambda i: (0, i, 0)),
            pl.BlockSpec((1, block_n, 32), lambda i: (1, i, 0)),
            pl.BlockSpec((block_n, 32), lambda i: (i, 0)),
            full((32, 32)), full((1, 32)),
        ],
        out_specs=pl.BlockSpec((block_n, 32), lambda i: (i, 0)),
        out_shape=jax.ShapeDtypeStruct((n_out, 32), jnp.float32),
    )(aggp, aggp, x, root, bias.reshape(1, 32))


def _y_stats_body(a0_ref, a1_ref, x_ref, root_ref, bias_ref, y_ref, stats_ref):
    y = a0_ref[0] + a1_ref[0] + x_ref[...] @ root_ref[...] + bias_ref[...]
    y_ref[...] = y

    @pl.when(pl.program_id(0) == 0)
    def _():
        stats_ref[...] = jnp.zeros_like(stats_ref)

    stats_ref[0:1, :] += jnp.sum(y, axis=0, keepdims=True)
    stats_ref[1:2, :] += jnp.sum(y * y, axis=0, keepdims=True)


def _bn_apply_body(y_ref, stats_ref, gamma_ref, beta_ref, out_ref, *, n):
    mean = stats_ref[0:1, :] / n
    var = stats_ref[1:2, :] / n - mean * mean
    yn = (y_ref[...] - mean) * jax.lax.rsqrt(var + 1e-5) * gamma_ref[...] \
        + beta_ref[...]
    out_ref[...] = _silu(yn)


def _update_bn(aggp, x, root, bias, gamma, beta, block_n=3000):
    n = x.shape[0]
    grid = n // block_n
    full = lambda shp: pl.BlockSpec(shp, lambda i: (0,) * len(shp))
    y, stats = pl.pallas_call(
        _y_stats_body,
        grid=(grid,),
        in_specs=[
            pl.BlockSpec((1, block_n, 32), lambda i: (0, i, 0)),
            pl.BlockSpec((1, block_n, 32), lambda i: (1, i, 0)),
            pl.BlockSpec((block_n, 32), lambda i: (i, 0)),
            full((32, 32)), full((1, 32)),
        ],
        out_specs=[
            pl.BlockSpec((block_n, 32), lambda i: (i, 0)),
            pl.BlockSpec((2, 32), lambda i: (0, 0)),
        ],
        out_shape=[
            jax.ShapeDtypeStruct((n, 32), jnp.float32),
            jax.ShapeDtypeStruct((2, 32), jnp.float32),
        ],
    )(aggp, aggp, x, root, bias.reshape(1, 32))
    return pl.pallas_call(
        functools.partial(_bn_apply_body, n=float(n)),
        grid=(grid,),
        in_specs=[
            pl.BlockSpec((block_n, 32), lambda i: (i, 0)),
            full((2, 32)), full((1, 32)), full((1, 32)),
        ],
        out_specs=pl.BlockSpec((block_n, 32), lambda i: (i, 0)),
        out_shape=jax.ShapeDtypeStruct((n, 32), jnp.float32),
    )(y, stats, gamma.reshape(1, 32), beta.reshape(1, 32))


def _head_body(pp_ref, molf_ref, wm0_ref, bm0_ref, gm_ref, btm_ref,
               wm1_ref, bm1_ref, wf0_ref, bf0_ref, wf1_ref, bf1_ref,
               wf2_ref, bf2_ref, wf3_ref, bf3_ref, out_ref):
    pool = pp_ref[0, 0:N_MOL_C, :] + pp_ref[1, 0:N_MOL_C, :]
    hm = molf_ref[...] @ wm0_ref[...] + bm0_ref[...]
    m = jnp.mean(hm, axis=0, keepdims=True)
    v = jnp.mean((hm - m) * (hm - m), axis=0, keepdims=True)
    hm = _silu((hm - m) * jax.lax.rsqrt(v + 1e-5) * gm_ref[...] + btm_ref[...])
    hm = _silu(hm @ wm1_ref[...] + bm1_ref[...])
    hg = jnp.concatenate([pool, hm], axis=1)
    hg = _silu(hg @ wf0_ref[...] + bf0_ref[...])
    hg = _silu(hg @ wf1_ref[...] + bf1_ref[...])
    hg = _silu(hg @ wf2_ref[...] + bf2_ref[...])
    out_ref[...] = hg @ wf3_ref[...] + bf3_ref[...]


def _head(poolp, mol_feats, p):
    wf3p = jnp.pad(p['Wf3'], ((0, 0), (0, 127)))
    bf3p = jnp.pad(p['bf3'].reshape(1, 1), ((0, 0), (0, 127)))
    args = (poolp, mol_feats,
            p['Wm0'], p['bm0'].reshape(1, 32),
            p['gamma_m'].reshape(1, 32), p['beta_m'].reshape(1, 32),
            p['Wm1'], p['bm1'].reshape(1, 32),
            p['Wf0'], p['bf0'].reshape(1, 128),
            p['Wf1'], p['bf1'].reshape(1, 128),
            p['Wf2'], p['bf2'].reshape(1, 64),
            wf3p, bf3p)
    in_specs = [pl.BlockSpec(a.shape, functools.partial(lambda nd: (0,) * nd,
                                                        a.ndim))
                for a in args]
    out = pl.pallas_call(
        _head_body,
        in_specs=in_specs,
        out_specs=pl.BlockSpec((N_MOL_C, 128), lambda: (0, 0)),
        out_shape=jax.ShapeDtypeStruct((N_MOL_C, 128), jnp.float32),
    )(*args)
    return out[:, :1]


def kernel(x, edge_attr, mol_feats, params, edge_index, batch):
    p = params
    n = x.shape[0]
    e = edge_attr.shape[0]

    wb2_1 = p['W1b'].reshape(1024, 32)
    bb_1 = p['b1b'].reshape(32, 32)
    wb2_2 = p['W2b'].reshape(1024, 32)
    bb_2 = p['b2b'].reshape(32, 32)

    src_pad = jnp.pad(edge_index[0], (0, _E_PAD - e))
    dst3 = jnp.pad(edge_index[1], (0, _E_PAD - e), constant_values=n) \
        .reshape(_NW, -1, _CH)
    batch3 = jnp.pad(batch, (0, _N_PAD - n), constant_values=N_MOL_C) \
        .reshape(_NW, -1, _CH)
    zeros_node = jnp.zeros((_NSEG_NODE, 32), jnp.float32)
    zeros_mol = jnp.zeros((_NSEG_MOL, 32), jnp.float32)

    def conv(h_nodes, wa, ba, wb2, bb):
        hs = _sc_gather(h_nodes, src_pad)
        msg = _msg(edge_attr, hs, wa, ba, wb2, bb)
        return _sc_scatter_add(msg, dst3, zeros_node, _NSEG_NODE)

    aggp = conv(x, p['W1a'], p['b1a'], wb2_1, bb_1)
    h = _update_bn(aggp, x, p['root0'], p['bias0'], p['gamma_gc'], p['beta_gc'])

    aggp = conv(h, p['W2a'], p['b2a'], wb2_2, bb_2)
    h2 = _update(aggp, h, p['root1'], p['bias1'], n)

    aggp = conv(h2, p['W2a'], p['b2a'], wb2_2, bb_2)
    h3p = _update(aggp, h2, p['root2'], p['bias2'], _N_PAD)

    poolp = _sc_scatter_add(h3p, batch3, zeros_mol, _NSEG_MOL)
    return _head(poolp, mol_feats, p)


# speedup vs baseline: 1.1555x; 1.0041x over previous
import functools

import jax
import jax.numpy as jnp
import numpy as np
from jax import lax
from jax.experimental import pallas as pl
from jax.experimental.pallas import tpu as pltpu
from jax.experimental.pallas import tpu_sc as plsc

N_NODES_C = 30000
N_EDGES_C = 60000
N_MOL_C = 1200

_NC, _NS, _NW = 2, 16, 32
_CH = 128

_E_PAD = 61440
_N_PAD = 32768
_NSEG_NODE = 30720
_NSEG_MOL = 1280

_R_NP = np.kron(np.eye(32, dtype=np.float32), np.ones((1, 32), dtype=np.float32))


def _sigmoid(v):
    return 1.0 / (1.0 + jnp.exp(-v))


def _silu(v):
    return v * _sigmoid(v)


def _mesh():
    return plsc.VectorSubcoreMesh(core_axis_name="c", subcore_axis_name="s",
                                  num_cores=_NC, num_subcores=_NS)


def _sc_gather(table, idx):
    e_pad = idx.shape[0]
    epw = e_pad // _NW
    nchunk = epw // _CH

    @functools.partial(
        pl.kernel,
        out_type=jax.ShapeDtypeStruct((e_pad, 32), jnp.float32),
        mesh=_mesh(),
        scratch_types=[
            pltpu.VMEM((epw,), jnp.int32),
            pltpu.VMEM((epw, 32), jnp.float32),
            pltpu.SemaphoreType.DMA,
            pltpu.SemaphoreType.DMA,
        ],
        compiler_params=pltpu.CompilerParams(use_tc_tiling_on_sc=False),
    )
    def k(table_hbm, idx_hbm, out_hbm, idx_v, rows_v, sem, sem_out):
        c = lax.axis_index("c")
        s = lax.axis_index("s")
        base = (c * _NS + s) * epw
        pltpu.sync_copy(idx_hbm.at[pl.ds(base, epw)], idx_v)
        cps = []
        for j in range(nchunk):
            cps.append(pltpu.async_copy(
                table_hbm.at[idx_v.at[pl.ds(j * _CH, _CH)]],
                rows_v.at[pl.ds(j * _CH, _CH)], sem))
        outs = []
        for j, cp in enumerate(cps):
            cp.wait()
            outs.append(pltpu.async_copy(
                rows_v.at[pl.ds(j * _CH, _CH)],
                out_hbm.at[pl.ds(base + j * _CH, _CH)], sem_out))
        for cp in outs:
            cp.wait()

    return k(table, idx)


def _sc_scatter_add(data, idx3, zeros, nseg_pad):
    nchunk = idx3.shape[1]
    epw = nchunk * _CH
    stripe = nseg_pad // _NS

    @functools.partial(
        pl.kernel,
        out_type=jax.ShapeDtypeStruct((_NC, nseg_pad, 32), jnp.float32),
        mesh=_mesh(),
        scratch_types=[
            pltpu.VMEM((nchunk, _CH), jnp.int32),
            pltpu.VMEM((epw, 32), jnp.float32),
            pltpu.VMEM_SHARED((nseg_pad, 32), jnp.float32),
            pltpu.SemaphoreType.DMA,
        ],
        compiler_params=pltpu.CompilerParams(use_tc_tiling_on_sc=False),
    )
    def k(data_hbm, idx_hbm, zeros_hbm, out_hbm, idx_v, data_v, acc_sh, sem):
        c = lax.axis_index("c")
        s = lax.axis_index("s")
        wid = c * _NS + s
        pltpu.sync_copy(zeros_hbm.at[pl.ds(s * stripe, stripe)],
                        acc_sh.at[pl.ds(s * stripe, stripe)])
        pltpu.sync_copy(idx_hbm.at[wid], idx_v)
        pltpu.sync_copy(data_hbm.at[pl.ds(wid * epw, epw)], data_v)
        plsc.subcore_barrier()
        cps = []
        for j in range(nchunk):
            cps.append(pltpu.async_copy(
                data_v.at[pl.ds(j * _CH, _CH)], acc_sh.at[idx_v.at[j]],
                sem, add=True))
        for cp in cps:
            cp.wait()
        plsc.subcore_barrier()
        pltpu.sync_copy(acc_sh.at[pl.ds(s * stripe, stripe)],
                        out_hbm.at[c, pl.ds(s * stripe, stripe)])

    return k(data, idx3, zeros)


def _msg_body(ea_ref, xs_ref, wa_ref, ba_ref, wb2_ref, bb_ref, r_ref,
              out_ref):
    h = jnp.maximum(ea_ref[...] @ wa_ref[...] + ba_ref[...], 0.0)
    xs = xs_ref[...]
    a = h @ r_ref[...]
    xs4 = jnp.concatenate([xs, xs, xs, xs], axis=1)
    b = jnp.concatenate([xs4] * 8, axis=1)
    z = a * b
    out_ref[...] = z @ wb2_ref[...] + xs @ bb_ref[...]


def _msg(edge_attr, x_src, wa, ba, wb2, bb, block_e=2000):
    e = edge_attr.shape[0]
    e_out = x_src.shape[0]
    grid = e // block_e
    full = lambda shp: pl.BlockSpec(shp, lambda i: (0, 0))
    return pl.pallas_call(
        _msg_body,
        grid=(grid,),
        in_specs=[
            pl.BlockSpec((block_e, 16), lambda i: (i, 0)),
            pl.BlockSpec((block_e, 32), lambda i: (i, 0)),
            full((16, 32)), full((1, 32)), full((1024, 32)), full((32, 32)),
            full((32, 1024)),
        ],
        out_specs=pl.BlockSpec((block_e, 32), lambda i: (i, 0)),
        out_shape=jax.ShapeDtypeStruct((e_out, 32), jnp.float32),
    )(edge_attr, x_src, wa, ba.reshape(1, 32), wb2, bb,
      jnp.asarray(_R_NP))


def _upd_body(a0_ref, a1_ref, x_ref, root_ref, bias_ref, out_ref):
    y = a0_ref[0] + a1_ref[0] + x_ref[...] @ root_ref[...] + bias_ref[...]
    out_ref[...] = _silu(y)


def _update(aggp, x, root, bias, n_out, block_n=3000):
    n = x.shape[0]
    grid = n // block_n
    full = lambda shp: pl.BlockSpec(shp, lambda i: (0,) * len(shp))
    return pl.pallas_call(
        _upd_body,
        grid=(grid,),
        in_specs=[
            pl.BlockSpec((1, block_n, 32), lambda i: (0, i, 0)),
            pl.BlockSpec((1, block_n, 32), lambda i: (1, i, 0)),
            pl.BlockSpec((block_n, 32), lambda i: (i, 0)),
            full((32, 32)), full((1, 32)),
        ],
        out_specs=pl.BlockSpec((block_n, 32), lambda i: (i, 0)),
        out_shape=jax.ShapeDtypeStruct((n_out, 32), jnp.float32),
    )(aggp, aggp, x, root, bias.reshape(1, 32))


def _y_stats_body(a0_ref, a1_ref, x_ref, root_ref, bias_ref, y_ref, stats_ref):
    y = a0_ref[0] + a1_ref[0] + x_ref[...] @ root_ref[...] + bias_ref[...]
    y_ref[...] = y

    @pl.when(pl.program_id(0) == 0)
    def _():
        stats_ref[...] = jnp.zeros_like(stats_ref)

    stats_ref[0:1, :] += jnp.sum(y, axis=0, keepdims=True)
    stats_ref[1:2, :] += jnp.sum(y * y, axis=0, keepdims=True)


def _bn_apply_body(y_ref, stats_ref, gamma_ref, beta_ref, out_ref, *, n):
    mean = stats_ref[0:1, :] / n
    var = stats_ref[1:2, :] / n - mean * mean
    yn = (y_ref[...] - mean) * jax.lax.rsqrt(var + 1e-5) * gamma_ref[...] \
        + beta_ref[...]
    out_ref[...] = _silu(yn)


def _update_bn(aggp, x, root, bias, gamma, beta, block_n=3000):
    n = x.shape[0]
    grid = n // block_n
    full = lambda shp: pl.BlockSpec(shp, lambda i: (0,) * len(shp))
    y, stats = pl.pallas_call(
        _y_stats_body,
        grid=(grid,),
        in_specs=[
            pl.BlockSpec((1, block_n, 32), lambda i: (0, i, 0)),
            pl.BlockSpec((1, block_n, 32), lambda i: (1, i, 0)),
            pl.BlockSpec((block_n, 32), lambda i: (i, 0)),
            full((32, 32)), full((1, 32)),
        ],
        out_specs=[
            pl.BlockSpec((block_n, 32), lambda i: (i, 0)),
            pl.BlockSpec((2, 32), lambda i: (0, 0)),
        ],
        out_shape=[
            jax.ShapeDtypeStruct((n, 32), jnp.float32),
            jax.ShapeDtypeStruct((2, 32), jnp.float32),
        ],
    )(aggp, aggp, x, root, bias.reshape(1, 32))
    return pl.pallas_call(
        functools.partial(_bn_apply_body, n=float(n)),
        grid=(grid,),
        in_specs=[
            pl.BlockSpec((block_n, 32), lambda i: (i, 0)),
            full((2, 32)), full((1, 32)), full((1, 32)),
        ],
        out_specs=pl.BlockSpec((block_n, 32), lambda i: (i, 0)),
        out_shape=jax.ShapeDtypeStruct((n, 32), jnp.float32),
    )(y, stats, gamma.reshape(1, 32), beta.reshape(1, 32))


def _head_body(pp_ref, molf_ref, wm0_ref, bm0_ref, gm_ref, btm_ref,
               wm1_ref, bm1_ref, wf0_ref, bf0_ref, wf1_ref, bf1_ref,
               wf2_ref, bf2_ref, wf3_ref, bf3_ref, out_ref):
    pool = pp_ref[0, 0:N_MOL_C, :] + pp_ref[1, 0:N_MOL_C, :]
    hm = molf_ref[...] @ wm0_ref[...] + bm0_ref[...]
    m = jnp.mean(hm, axis=0, keepdims=True)
    v = jnp.mean((hm - m) * (hm - m), axis=0, keepdims=True)
    hm = _silu((hm - m) * jax.lax.rsqrt(v + 1e-5) * gm_ref[...] + btm_ref[...])
    hm = _silu(hm @ wm1_ref[...] + bm1_ref[...])
    hg = jnp.concatenate([pool, hm], axis=1)
    hg = _silu(hg @ wf0_ref[...] + bf0_ref[...])
    hg = _silu(hg @ wf1_ref[...] + bf1_ref[...])
    hg = _silu(hg @ wf2_ref[...] + bf2_ref[...])
    out_ref[...] = hg @ wf3_ref[...] + bf3_ref[...]


def _head(poolp, mol_feats, p):
    wf3p = jnp.pad(p['Wf3'], ((0, 0), (0, 127)))
    bf3p = jnp.pad(p['bf3'].reshape(1, 1), ((0, 0), (0, 127)))
    args = (poolp, mol_feats,
            p['Wm0'], p['bm0'].reshape(1, 32),
            p['gamma_m'].reshape(1, 32), p['beta_m'].reshape(1, 32),
            p['Wm1'], p['bm1'].reshape(1, 32),
            p['Wf0'], p['bf0'].reshape(1, 128),
            p['Wf1'], p['bf1'].reshape(1, 128),
            p['Wf2'], p['bf2'].reshape(1, 64),
            wf3p, bf3p)
    in_specs = [pl.BlockSpec(a.shape, functools.partial(lambda nd: (0,) * nd,
                                                        a.ndim))
                for a in args]
    out = pl.pallas_call(
        _head_body,
        in_specs=in_specs,
        out_specs=pl.BlockSpec((N_MOL_C, 128), lambda: (0, 0)),
        out_shape=jax.ShapeDtypeStruct((N_MOL_C, 128), jnp.float32),
    )(*args)
    return out[:, :1]


def kernel(x, edge_attr, mol_feats, params, edge_index, batch):
    p = params
    n = x.shape[0]
    e = edge_attr.shape[0]

    wb2_1 = p['W1b'].reshape(1024, 32)
    bb_1 = p['b1b'].reshape(32, 32)
    wb2_2 = p['W2b'].reshape(1024, 32)
    bb_2 = p['b2b'].reshape(32, 32)

    src_pad = jnp.pad(edge_index[0], (0, _E_PAD - e))
    dst3 = jnp.pad(edge_index[1], (0, _E_PAD - e), constant_values=n) \
        .reshape(_NW, -1, _CH)
    batch3 = jnp.pad(batch, (0, _N_PAD - n), constant_values=N_MOL_C) \
        .reshape(_NW, -1, _CH)
    zeros_node = jnp.zeros((_NSEG_NODE, 32), jnp.float32)
    zeros_mol = jnp.zeros((_NSEG_MOL, 32), jnp.float32)

    def conv(h_nodes, wa, ba, wb2, bb):
        hs = _sc_gather(h_nodes, src_pad)
        msg = _msg(edge_attr, hs, wa, ba, wb2, bb)
        return _sc_scatter_add(msg, dst3, zeros_node, _NSEG_NODE)

    aggp = conv(x, p['W1a'], p['b1a'], wb2_1, bb_1)
    h = _update_bn(aggp, x, p['root0'], p['bias0'], p['gamma_gc'], p['beta_gc'])

    aggp = conv(h, p['W2a'], p['b2a'], wb2_2, bb_2)
    h2 = _update(aggp, h, p['root1'], p['bias1'], n)

    aggp = conv(h2, p['W2a'], p['b2a'], wb2_2, bb_2)
    h3p = _update(aggp, h2, p['root2'], p['bias2'], _N_PAD)

    poolp = _sc_scatter_add(h3p, batch3, zeros_mol, _NSEG_MOL)
    return _head(poolp, mol_feats, p)
